# initial kernel scaffold (unmeasured)
import jax
import jax.numpy as jnp
from jax import lax
from jax.experimental import pallas as pl
from jax.experimental.pallas import tpu as pltpu

N_DEV = 4


def kernel(x, w_mat, scale_x, scale_w):
    x8 = x.astype(jnp.float8_e5m2)
    w8 = w_mat.astype(jnp.float8_e5m2)
    m_per, k = x8.shape
    n_local = w8.shape[1]

    def body(x_ref, w_ref, sx_ref, sw_ref, out_ref, comm_ref, send_sems, recv_sems):
        my = lax.axis_index("i")
        right = (my + 1) % N_DEV
        left = (my + N_DEV - 1) % N_DEV

        barrier = pltpu.get_barrier_semaphore()
        for nbr in (left, right):
            pl.semaphore_signal(
                barrier, inc=1, device_id=(nbr,),
                device_id_type=pl.DeviceIdType.MESH,
            )
        pl.semaphore_wait(barrier, 2)

        scale = sx_ref[0] * sw_ref[0]

        def compute(chunk, origin):
            acc = jnp.dot(chunk, w_ref[...], preferred_element_type=jnp.float32)
            y = acc * scale
            z = y / (1.0 + jnp.exp(-jnp.clip(y, -60.0, 60.0)))
            out_ref[pl.ds(origin * m_per, m_per), :] = z

        compute(x_ref[...], my)

        for h in range(N_DEV - 1):
            src = x_ref if h == 0 else comm_ref.at[h - 1]
            rdma = pltpu.make_async_remote_copy(
                src_ref=src,
                dst_ref=comm_ref.at[h],
                send_sem=send_sems.at[h],
                recv_sem=recv_sems.at[h],
                device_id=(right,),
                device_id_type=pl.DeviceIdType.MESH,
            )
            rdma.start()
            rdma.wait()
            origin = (my + N_DEV - 1 - h) % N_DEV
            compute(comm_ref[h], origin)

    return pl.pallas_call(
        body,
        out_shape=jax.ShapeDtypeStruct((N_DEV * m_per, n_local), jnp.float32),
        in_specs=[
            pl.BlockSpec(memory_space=pltpu.VMEM),
            pl.BlockSpec(memory_space=pltpu.VMEM),
            pl.BlockSpec(memory_space=pltpu.SMEM),
            pl.BlockSpec(memory_space=pltpu.SMEM),
        ],
        out_specs=pl.BlockSpec(memory_space=pltpu.VMEM),
        scratch_shapes=[
            pltpu.VMEM((N_DEV - 1, m_per, k), jnp.float8_e5m2),
            pltpu.SemaphoreType.DMA((N_DEV - 1,)),
            pltpu.SemaphoreType.DMA((N_DEV - 1,)),
        ],
        compiler_params=pltpu.CompilerParams(collective_id=0),
    )(x8, w8, scale_x, scale_w)


# baseline (device time: 247824 ns/iter reference)
import jax
import jax.numpy as jnp
from jax import lax
from jax.experimental import pallas as pl
from jax.experimental.pallas import tpu as pltpu

N_DEV = 4


def kernel(x, w_mat, scale_x, scale_w):
    x8 = x.astype(jnp.float8_e5m2)
    w8 = w_mat.astype(jnp.float8_e5m2)
    m_per, k = x8.shape
    n_local = w8.shape[1]

    def body(x_ref, w_ref, sx_ref, sw_ref, out_ref, comm_ref, send_sems, recv_sems):
        my = lax.axis_index("i")
        right = (my + 1) % N_DEV
        left = (my + N_DEV - 1) % N_DEV

        barrier = pltpu.get_barrier_semaphore()
        for nbr in (left, right):
            pl.semaphore_signal(
                barrier, inc=1, device_id=(nbr,),
                device_id_type=pl.DeviceIdType.MESH,
            )
        pl.semaphore_wait(barrier, 2)

        scale = sx_ref[0] * sw_ref[0]

        def compute(chunk, origin):
            acc = jnp.dot(chunk, w_ref[...], preferred_element_type=jnp.float32)
            y = acc * scale
            z = y / (1.0 + jnp.exp(-jnp.clip(y, -60.0, 60.0)))
            out_ref[pl.ds(origin * m_per, m_per), :] = z

        compute(x_ref[...], my)

        for h in range(N_DEV - 1):
            src = x_ref if h == 0 else comm_ref.at[h - 1]
            rdma = pltpu.make_async_remote_copy(
                src_ref=src,
                dst_ref=comm_ref.at[h],
                send_sem=send_sems.at[h],
                recv_sem=recv_sems.at[h],
                device_id=(right,),
                device_id_type=pl.DeviceIdType.MESH,
            )
            rdma.start()
            rdma.wait()
            origin = (my + N_DEV - 1 - h) % N_DEV
            compute(comm_ref[h], origin)

    return pl.pallas_call(
        body,
        out_shape=jax.ShapeDtypeStruct((N_DEV * m_per, n_local), jnp.float32),
        in_specs=[
            pl.BlockSpec(memory_space=pltpu.VMEM),
            pl.BlockSpec(memory_space=pltpu.VMEM),
            pl.BlockSpec(memory_space=pltpu.SMEM),
            pl.BlockSpec(memory_space=pltpu.SMEM),
        ],
        out_specs=pl.BlockSpec(memory_space=pltpu.VMEM),
        scratch_shapes=[
            pltpu.VMEM((N_DEV - 1, m_per, k), jnp.float8_e5m2),
            pltpu.SemaphoreType.DMA((N_DEV - 1,)),
            pltpu.SemaphoreType.DMA((N_DEV - 1,)),
        ],
        compiler_params=pltpu.CompilerParams(
            collective_id=0, vmem_limit_bytes=100 * 1024 * 1024
        ),
    )(x8, w8, scale_x, scale_w)


# device time: 147889 ns/iter; 1.6757x vs baseline; 1.6757x over previous
import jax
import jax.numpy as jnp
from jax import lax
from jax.experimental import pallas as pl
from jax.experimental.pallas import tpu as pltpu

N_DEV = 4
N_HOP = N_DEV - 1


def kernel(x, w_mat, scale_x, scale_w):
    x8 = x.astype(jnp.float8_e5m2)
    w8 = w_mat.astype(jnp.float8_e5m2)
    m_per, k = x8.shape
    n_local = w_mat.shape[1]
    half = m_per // 2

    def body(x_ref, w8_ref, sx_ref, sw_ref, out_ref,
             comm_r, comm_l, ssem_r, rsem_r, ssem_l, rsem_l):
        my = lax.axis_index("i")
        right = (my + 1) % N_DEV
        left = (my + N_DEV - 1) % N_DEV

        barrier = pltpu.get_barrier_semaphore()
        for nbr in (left, right):
            pl.semaphore_signal(
                barrier, inc=1, device_id=(nbr,),
                device_id_type=pl.DeviceIdType.MESH,
            )
        pl.semaphore_wait(barrier, 2)

        def send(src, dst, ssem, rsem, dev):
            rdma = pltpu.make_async_remote_copy(
                src_ref=src, dst_ref=dst, send_sem=ssem, recv_sem=rsem,
                device_id=(dev,), device_id_type=pl.DeviceIdType.MESH,
            )
            rdma.start()

        send(x_ref.at[0:half], comm_r.at[0], ssem_r.at[0], rsem_r.at[0], right)
        send(x_ref.at[half:m_per], comm_l.at[0], ssem_l.at[0], rsem_l.at[0], left)

        scale = sx_ref[0] * sw_ref[0]

        def compute(chunk, out_row):
            acc = jnp.dot(chunk, w8_ref[...], preferred_element_type=jnp.float32)
            y = acc * scale
            z = y / (1.0 + jnp.exp(-jnp.clip(y, -60.0, 60.0)))
            out_ref[pl.ds(out_row, chunk.shape[0]), :] = z

        compute(x_ref[...], my * m_per)

        for h in range(N_HOP):
            recv_r = pltpu.make_async_remote_copy(
                src_ref=comm_r.at[h], dst_ref=comm_r.at[h],
                send_sem=ssem_r.at[h], recv_sem=rsem_r.at[h],
                device_id=(right,), device_id_type=pl.DeviceIdType.MESH,
            )
            recv_r.wait_recv()
            recv_l = pltpu.make_async_remote_copy(
                src_ref=comm_l.at[h], dst_ref=comm_l.at[h],
                send_sem=ssem_l.at[h], recv_sem=rsem_l.at[h],
                device_id=(left,), device_id_type=pl.DeviceIdType.MESH,
            )
            recv_l.wait_recv()
            if h < N_HOP - 1:
                send(comm_r.at[h], comm_r.at[h + 1],
                     ssem_r.at[h + 1], rsem_r.at[h + 1], right)
                send(comm_l.at[h], comm_l.at[h + 1],
                     ssem_l.at[h + 1], rsem_l.at[h + 1], left)
            origin_r = (my + N_DEV - 1 - h) % N_DEV
            origin_l = (my + h + 1) % N_DEV
            compute(comm_r[h], origin_r * m_per)
            compute(comm_l[h], origin_l * m_per + half)

        for h in range(N_HOP):
            for (sl, ssem, rsem, dev) in (
                (comm_r, ssem_r, rsem_r, right),
                (comm_l, ssem_l, rsem_l, left),
            ):
                done = pltpu.make_async_remote_copy(
                    src_ref=sl.at[h], dst_ref=sl.at[h],
                    send_sem=ssem.at[h], recv_sem=rsem.at[h],
                    device_id=(dev,), device_id_type=pl.DeviceIdType.MESH,
                )
                done.wait_send()

    return pl.pallas_call(
        body,
        out_shape=jax.ShapeDtypeStruct((N_DEV * m_per, n_local), jnp.float32),
        in_specs=[
            pl.BlockSpec(memory_space=pltpu.VMEM),
            pl.BlockSpec(memory_space=pltpu.VMEM),
            pl.BlockSpec(memory_space=pltpu.SMEM),
            pl.BlockSpec(memory_space=pltpu.SMEM),
        ],
        out_specs=pl.BlockSpec(memory_space=pltpu.VMEM),
        scratch_shapes=[
            pltpu.VMEM((N_HOP, half, k), jnp.float8_e5m2),
            pltpu.VMEM((N_HOP, half, k), jnp.float8_e5m2),
            pltpu.SemaphoreType.DMA((N_HOP,)),
            pltpu.SemaphoreType.DMA((N_HOP,)),
            pltpu.SemaphoreType.DMA((N_HOP,)),
            pltpu.SemaphoreType.DMA((N_HOP,)),
        ],
        compiler_params=pltpu.CompilerParams(
            collective_id=0, vmem_limit_bytes=100 * 1024 * 1024
        ),
    )(x8, w8, scale_x, scale_w)


# device time: 139371 ns/iter; 1.7782x vs baseline; 1.0611x over previous
import jax
import jax.numpy as jnp
from jax import lax
from jax.experimental import pallas as pl
from jax.experimental.pallas import tpu as pltpu

N_DEV = 4
N_HOP = N_DEV - 1
N_STAGE = 3


def kernel(x, w_mat, scale_x, scale_w):
    x8 = x.astype(jnp.float8_e5m2)
    w8 = w_mat.astype(jnp.float8_e5m2)
    m_per, k = x8.shape
    n_local = w_mat.shape[1]
    half = m_per // 2

    def body(x_ref, w8_ref, sx_ref, sw_ref, out_hbm,
             comm_r, comm_l, stage, ssem_r, rsem_r, ssem_l, rsem_l, osem):
        my = lax.axis_index("i")
        right = (my + 1) % N_DEV
        left = (my + N_DEV - 1) % N_DEV

        barrier = pltpu.get_barrier_semaphore()
        for nbr in (left, right):
            pl.semaphore_signal(
                barrier, inc=1, device_id=(nbr,),
                device_id_type=pl.DeviceIdType.MESH,
            )
        pl.semaphore_wait(barrier, 2)

        def send(src, dst, ssem, rsem, dev):
            rdma = pltpu.make_async_remote_copy(
                src_ref=src, dst_ref=dst, send_sem=ssem, recv_sem=rsem,
                device_id=(dev,), device_id_type=pl.DeviceIdType.MESH,
            )
            rdma.start()

        send(x_ref.at[0:half], comm_r.at[0], ssem_r.at[0], rsem_r.at[0], right)
        send(x_ref.at[half:m_per], comm_l.at[0], ssem_l.at[0], rsem_l.at[0], left)

        scale = sx_ref[0] * sw_ref[0]

        pending = {}
        blk = [0]

        def compute(chunk, out_row):
            s = blk[0] % N_STAGE
            blk[0] += 1
            if s in pending:
                pending.pop(s).wait()
            acc = jnp.dot(chunk, w8_ref[...], preferred_element_type=jnp.float32)
            y = acc * scale
            stage[s] = y / (1.0 + jnp.exp(-jnp.clip(y, -60.0, 60.0)))
            cp = pltpu.make_async_copy(
                stage.at[s], out_hbm.at[pl.ds(out_row, half)], osem.at[s]
            )
            cp.start()
            pending[s] = cp

        compute(x_ref[0:half], my * m_per)
        compute(x_ref[half:m_per], my * m_per + half)

        for h in range(N_HOP):
            recv_r = pltpu.make_async_remote_copy(
                src_ref=comm_r.at[h], dst_ref=comm_r.at[h],
                send_sem=ssem_r.at[h], recv_sem=rsem_r.at[h],
                device_id=(right,), device_id_type=pl.DeviceIdType.MESH,
            )
            recv_r.wait_recv()
            recv_l = pltpu.make_async_remote_copy(
                src_ref=comm_l.at[h], dst_ref=comm_l.at[h],
                send_sem=ssem_l.at[h], recv_sem=rsem_l.at[h],
                device_id=(left,), device_id_type=pl.DeviceIdType.MESH,
            )
            recv_l.wait_recv()
            if h < N_HOP - 1:
                send(comm_r.at[h], comm_r.at[h + 1],
                     ssem_r.at[h + 1], rsem_r.at[h + 1], right)
                send(comm_l.at[h], comm_l.at[h + 1],
                     ssem_l.at[h + 1], rsem_l.at[h + 1], left)
            origin_r = (my + N_DEV - 1 - h) % N_DEV
            origin_l = (my + h + 1) % N_DEV
            compute(comm_r[h], origin_r * m_per)
            compute(comm_l[h], origin_l * m_per + half)

        for cp in pending.values():
            cp.wait()
        for h in range(N_HOP):
            for (sl, ssem, rsem, dev) in (
                (comm_r, ssem_r, rsem_r, right),
                (comm_l, ssem_l, rsem_l, left),
            ):
                done = pltpu.make_async_remote_copy(
                    src_ref=sl.at[h], dst_ref=sl.at[h],
                    send_sem=ssem.at[h], recv_sem=rsem.at[h],
                    device_id=(dev,), device_id_type=pl.DeviceIdType.MESH,
                )
                done.wait_send()

    return pl.pallas_call(
        body,
        out_shape=jax.ShapeDtypeStruct((N_DEV * m_per, n_local), jnp.float32),
        in_specs=[
            pl.BlockSpec(memory_space=pltpu.VMEM),
            pl.BlockSpec(memory_space=pltpu.VMEM),
            pl.BlockSpec(memory_space=pltpu.SMEM),
            pl.BlockSpec(memory_space=pltpu.SMEM),
        ],
        out_specs=pl.BlockSpec(memory_space=pl.ANY),
        scratch_shapes=[
            pltpu.VMEM((N_HOP, half, k), jnp.float8_e5m2),
            pltpu.VMEM((N_HOP, half, k), jnp.float8_e5m2),
            pltpu.VMEM((N_STAGE, half, n_local), jnp.float32),
            pltpu.SemaphoreType.DMA((N_HOP,)),
            pltpu.SemaphoreType.DMA((N_HOP,)),
            pltpu.SemaphoreType.DMA((N_HOP,)),
            pltpu.SemaphoreType.DMA((N_HOP,)),
            pltpu.SemaphoreType.DMA((N_STAGE,)),
        ],
        compiler_params=pltpu.CompilerParams(
            collective_id=0, vmem_limit_bytes=100 * 1024 * 1024
        ),
    )(x8, w8, scale_x, scale_w)


# device time: 129444 ns/iter; 1.9145x vs baseline; 1.0767x over previous
import jax
import jax.numpy as jnp
from jax import lax
from jax.experimental import pallas as pl
from jax.experimental.pallas import tpu as pltpu

N_DEV = 4
N_HOP = N_DEV - 1
N_PIECE = 2
N_STAGE = 4


def kernel(x, w_mat, scale_x, scale_w):
    x8 = x.astype(jnp.float8_e5m2)
    w8 = w_mat.astype(jnp.float8_e5m2)
    m_per, k = x8.shape
    n_local = w_mat.shape[1]
    half = m_per // 2
    q = half // N_PIECE

    def body(x_ref, w8_ref, sx_ref, sw_ref, out_hbm,
             comm_r, comm_l, stage, ssem_r, rsem_r, ssem_l, rsem_l, osem):
        my = lax.axis_index("i")
        right = (my + 1) % N_DEV
        left = (my + N_DEV - 1) % N_DEV

        barrier = pltpu.get_barrier_semaphore()
        for nbr in (left, right):
            pl.semaphore_signal(
                barrier, inc=1, device_id=(nbr,),
                device_id_type=pl.DeviceIdType.MESH,
            )
        pl.semaphore_wait(barrier, 2)

        def send(src, dst, ssem, rsem, dev):
            rdma = pltpu.make_async_remote_copy(
                src_ref=src, dst_ref=dst, send_sem=ssem, recv_sem=rsem,
                device_id=(dev,), device_id_type=pl.DeviceIdType.MESH,
            )
            rdma.start()

        for p in range(N_PIECE):
            send(x_ref.at[p * q:(p + 1) * q], comm_r.at[0, p * q:(p + 1) * q],
                 ssem_r.at[0, p], rsem_r.at[0, p], right)
        for p in range(N_PIECE):
            send(x_ref.at[half + p * q:half + (p + 1) * q],
                 comm_l.at[0, p * q:(p + 1) * q],
                 ssem_l.at[0, p], rsem_l.at[0, p], left)

        scale = sx_ref[0] * sw_ref[0]

        pending = {}
        blk = [0]

        def compute(chunk, out_row):
            s = blk[0] % N_STAGE
            blk[0] += 1
            if s in pending:
                pending.pop(s).wait()
            acc = jnp.dot(chunk, w8_ref[...], preferred_element_type=jnp.float32)
            y = acc * scale
            stage[s] = y / (1.0 + jnp.exp(-jnp.clip(y, -60.0, 60.0)))
            cp = pltpu.make_async_copy(
                stage.at[s], out_hbm.at[pl.ds(out_row, q)], osem.at[s]
            )
            cp.start()
            pending[s] = cp

        def compute_own(p):
            compute(x_ref[p * q:(p + 1) * q], my * m_per + p * q)

        compute_own(0)
        compute_own(1)
        own_next = [2]

        def hop_piece(h, p):
            recv_r = pltpu.make_async_remote_copy(
                src_ref=comm_r.at[h, p * q:(p + 1) * q],
                dst_ref=comm_r.at[h, p * q:(p + 1) * q],
                send_sem=ssem_r.at[h, p], recv_sem=rsem_r.at[h, p],
                device_id=(right,), device_id_type=pl.DeviceIdType.MESH,
            )
            recv_r.wait_recv()
            if h < N_HOP - 1:
                send(comm_r.at[h, p * q:(p + 1) * q],
                     comm_r.at[h + 1, p * q:(p + 1) * q],
                     ssem_r.at[h + 1, p], rsem_r.at[h + 1, p], right)
            recv_l = pltpu.make_async_remote_copy(
                src_ref=comm_l.at[h, p * q:(p + 1) * q],
                dst_ref=comm_l.at[h, p * q:(p + 1) * q],
                send_sem=ssem_l.at[h, p], recv_sem=rsem_l.at[h, p],
                device_id=(left,), device_id_type=pl.DeviceIdType.MESH,
            )
            recv_l.wait_recv()
            if h < N_HOP - 1:
                send(comm_l.at[h, p * q:(p + 1) * q],
                     comm_l.at[h + 1, p * q:(p + 1) * q],
                     ssem_l.at[h + 1, p], rsem_l.at[h + 1, p], left)
            origin_r = (my + N_DEV - 1 - h) % N_DEV
            origin_l = (my + h + 1) % N_DEV
            compute(comm_r[h, p * q:(p + 1) * q], origin_r * m_per + p * q)
            compute(comm_l[h, p * q:(p + 1) * q],
                    origin_l * m_per + half + p * q)

        for h in range(N_HOP):
            for p in range(N_PIECE):
                hop_piece(h, p)
                if own_next[0] < N_PIECE * 2:
                    compute_own(own_next[0])
                    own_next[0] += 1

        for cp in pending.values():
            cp.wait()
        for h in range(N_HOP):
            for p in range(N_PIECE):
                for (sl, ssem, rsem, dev) in (
                    (comm_r, ssem_r, rsem_r, right),
                    (comm_l, ssem_l, rsem_l, left),
                ):
                    done = pltpu.make_async_remote_copy(
                        src_ref=sl.at[h, p * q:(p + 1) * q],
                        dst_ref=sl.at[h, p * q:(p + 1) * q],
                        send_sem=ssem.at[h, p], recv_sem=rsem.at[h, p],
                        device_id=(dev,), device_id_type=pl.DeviceIdType.MESH,
                    )
                    done.wait_send()

    return pl.pallas_call(
        body,
        out_shape=jax.ShapeDtypeStruct((N_DEV * m_per, n_local), jnp.float32),
        in_specs=[
            pl.BlockSpec(memory_space=pltpu.VMEM),
            pl.BlockSpec(memory_space=pltpu.VMEM),
            pl.BlockSpec(memory_space=pltpu.SMEM),
            pl.BlockSpec(memory_space=pltpu.SMEM),
        ],
        out_specs=pl.BlockSpec(memory_space=pl.ANY),
        scratch_shapes=[
            pltpu.VMEM((N_HOP, half, k), jnp.float8_e5m2),
            pltpu.VMEM((N_HOP, half, k), jnp.float8_e5m2),
            pltpu.VMEM((N_STAGE, q, n_local), jnp.float32),
            pltpu.SemaphoreType.DMA((N_HOP, N_PIECE)),
            pltpu.SemaphoreType.DMA((N_HOP, N_PIECE)),
            pltpu.SemaphoreType.DMA((N_HOP, N_PIECE)),
            pltpu.SemaphoreType.DMA((N_HOP, N_PIECE)),
            pltpu.SemaphoreType.DMA((N_STAGE,)),
        ],
        compiler_params=pltpu.CompilerParams(
            collective_id=0, vmem_limit_bytes=100 * 1024 * 1024
        ),
    )(x8, w8, scale_x, scale_w)


# device time: 112986 ns/iter; 2.1934x vs baseline; 1.1457x over previous
import jax
import jax.numpy as jnp
from jax import lax
from jax.experimental import pallas as pl
from jax.experimental.pallas import tpu as pltpu

N_DEV = 4
N_HOP = N_DEV - 1
N_PIECE = 2
N_STAGE = 4


def kernel(x, w_mat, scale_x, scale_w):
    x8 = x.astype(jnp.float8_e5m2)
    m_per, k = x8.shape
    n_local = w_mat.shape[1]
    half = m_per // 2
    q = half // N_PIECE

    W_CHUNK = 512
    N_WSTAGE = 2

    def body(x_ref, w_hbm, sx_ref, sw_ref, out_hbm,
             comm_r, comm_l, stage, w8_ref, wstage, ssem_r, rsem_r,
             ssem_l, rsem_l, osem, wsem):
        my = lax.axis_index("i")
        right = (my + 1) % N_DEV
        left = (my + N_DEV - 1) % N_DEV

        barrier = pltpu.get_barrier_semaphore()
        for nbr in (left, right):
            pl.semaphore_signal(
                barrier, inc=1, device_id=(nbr,),
                device_id_type=pl.DeviceIdType.MESH,
            )
        pl.semaphore_wait(barrier, 2)

        def send(src, dst, ssem, rsem, dev):
            rdma = pltpu.make_async_remote_copy(
                src_ref=src, dst_ref=dst, send_sem=ssem, recv_sem=rsem,
                device_id=(dev,), device_id_type=pl.DeviceIdType.MESH,
            )
            rdma.start()

        for p in range(N_PIECE):
            send(x_ref.at[p * q:(p + 1) * q], comm_r.at[0, p * q:(p + 1) * q],
                 ssem_r.at[0, p], rsem_r.at[0, p], right)
        for p in range(N_PIECE):
            send(x_ref.at[half + p * q:half + (p + 1) * q],
                 comm_l.at[0, p * q:(p + 1) * q],
                 ssem_l.at[0, p], rsem_l.at[0, p], left)

        wcps = {}
        n_wchunk = k // W_CHUNK
        for j in range(min(N_WSTAGE, n_wchunk)):
            cp = pltpu.make_async_copy(
                w_hbm.at[pl.ds(j * W_CHUNK, W_CHUNK)], wstage.at[j % N_WSTAGE],
                wsem.at[j % N_WSTAGE],
            )
            cp.start()
            wcps[j % N_WSTAGE] = cp
        for j in range(n_wchunk):
            s = j % N_WSTAGE
            wcps.pop(s).wait()
            nxt = j + N_WSTAGE
            if nxt < n_wchunk:
                w8_ref[pl.ds(j * W_CHUNK, W_CHUNK)] = (
                    wstage[s].astype(jnp.float8_e5m2))
                cp = pltpu.make_async_copy(
                    w_hbm.at[pl.ds(nxt * W_CHUNK, W_CHUNK)], wstage.at[s],
                    wsem.at[s],
                )
                cp.start()
                wcps[s] = cp
            else:
                w8_ref[pl.ds(j * W_CHUNK, W_CHUNK)] = (
                    wstage[s].astype(jnp.float8_e5m2))

        scale = sx_ref[0] * sw_ref[0]

        pending = {}
        blk = [0]

        def compute(chunk, out_row):
            s = blk[0] % N_STAGE
            blk[0] += 1
            if s in pending:
                pending.pop(s).wait()
            acc = jnp.dot(chunk, w8_ref[...], preferred_element_type=jnp.float32)
            y = acc * scale
            stage[s] = y / (1.0 + jnp.exp(-jnp.clip(y, -60.0, 60.0)))
            cp = pltpu.make_async_copy(
                stage.at[s], out_hbm.at[pl.ds(out_row, q)], osem.at[s]
            )
            cp.start()
            pending[s] = cp

        def compute_own(p):
            compute(x_ref[p * q:(p + 1) * q], my * m_per + p * q)

        compute_own(0)
        compute_own(1)
        own_next = [2]

        def hop_piece(h, p):
            recv_r = pltpu.make_async_remote_copy(
                src_ref=comm_r.at[h, p * q:(p + 1) * q],
                dst_ref=comm_r.at[h, p * q:(p + 1) * q],
                send_sem=ssem_r.at[h, p], recv_sem=rsem_r.at[h, p],
                device_id=(right,), device_id_type=pl.DeviceIdType.MESH,
            )
            recv_r.wait_recv()
            if h < N_HOP - 1:
                send(comm_r.at[h, p * q:(p + 1) * q],
                     comm_r.at[h + 1, p * q:(p + 1) * q],
                     ssem_r.at[h + 1, p], rsem_r.at[h + 1, p], right)
            recv_l = pltpu.make_async_remote_copy(
                src_ref=comm_l.at[h, p * q:(p + 1) * q],
                dst_ref=comm_l.at[h, p * q:(p + 1) * q],
                send_sem=ssem_l.at[h, p], recv_sem=rsem_l.at[h, p],
                device_id=(left,), device_id_type=pl.DeviceIdType.MESH,
            )
            recv_l.wait_recv()
            if h < N_HOP - 1:
                send(comm_l.at[h, p * q:(p + 1) * q],
                     comm_l.at[h + 1, p * q:(p + 1) * q],
                     ssem_l.at[h + 1, p], rsem_l.at[h + 1, p], left)
            origin_r = (my + N_DEV - 1 - h) % N_DEV
            origin_l = (my + h + 1) % N_DEV
            compute(comm_r[h, p * q:(p + 1) * q], origin_r * m_per + p * q)
            compute(comm_l[h, p * q:(p + 1) * q],
                    origin_l * m_per + half + p * q)

        for h in range(N_HOP):
            for p in range(N_PIECE):
                hop_piece(h, p)
                if own_next[0] < N_PIECE * 2:
                    compute_own(own_next[0])
                    own_next[0] += 1

        for cp in pending.values():
            cp.wait()
        for h in range(N_HOP):
            for p in range(N_PIECE):
                for (sl, ssem, rsem, dev) in (
                    (comm_r, ssem_r, rsem_r, right),
                    (comm_l, ssem_l, rsem_l, left),
                ):
                    done = pltpu.make_async_remote_copy(
                        src_ref=sl.at[h, p * q:(p + 1) * q],
                        dst_ref=sl.at[h, p * q:(p + 1) * q],
                        send_sem=ssem.at[h, p], recv_sem=rsem.at[h, p],
                        device_id=(dev,), device_id_type=pl.DeviceIdType.MESH,
                    )
                    done.wait_send()

    return pl.pallas_call(
        body,
        out_shape=jax.ShapeDtypeStruct((N_DEV * m_per, n_local), jnp.float32),
        in_specs=[
            pl.BlockSpec(memory_space=pltpu.VMEM),
            pl.BlockSpec(memory_space=pl.ANY),
            pl.BlockSpec(memory_space=pltpu.SMEM),
            pl.BlockSpec(memory_space=pltpu.SMEM),
        ],
        out_specs=pl.BlockSpec(memory_space=pl.ANY),
        scratch_shapes=[
            pltpu.VMEM((N_HOP, half, k), jnp.float8_e5m2),
            pltpu.VMEM((N_HOP, half, k), jnp.float8_e5m2),
            pltpu.VMEM((N_STAGE, q, n_local), jnp.float32),
            pltpu.VMEM((k, n_local), jnp.float8_e5m2),
            pltpu.VMEM((2, 512, n_local), jnp.float32),
            pltpu.SemaphoreType.DMA((N_HOP, N_PIECE)),
            pltpu.SemaphoreType.DMA((N_HOP, N_PIECE)),
            pltpu.SemaphoreType.DMA((N_HOP, N_PIECE)),
            pltpu.SemaphoreType.DMA((N_HOP, N_PIECE)),
            pltpu.SemaphoreType.DMA((N_STAGE,)),
            pltpu.SemaphoreType.DMA((2,)),
        ],
        compiler_params=pltpu.CompilerParams(
            collective_id=0, vmem_limit_bytes=100 * 1024 * 1024
        ),
    )(x8, w_mat, scale_x, scale_w)


# device time: 107313 ns/iter; 2.3094x vs baseline; 1.0529x over previous
import jax
import jax.numpy as jnp
from jax import lax
from jax.experimental import pallas as pl
from jax.experimental.pallas import tpu as pltpu

N_DEV = 4
N_HOP = N_DEV - 1
N_PIECE = 2
N_STAGE = 4


def kernel(x, w_mat, scale_x, scale_w):
    m_per, k = x.shape
    n_local = w_mat.shape[1]
    half = m_per // 2
    q = half // N_PIECE

    W_CHUNK = 512
    N_WSTAGE = 2

    X_CHUNK = 256
    N_XSTAGE = 2

    def body(x_hbm, w_hbm, sx_ref, sw_ref, out_hbm,
             comm_r, comm_l, stage, w8_ref, wstage, x8_ref, xstage,
             ssem_r, rsem_r, ssem_l, rsem_l, osem, wsem, xsem):
        my = lax.axis_index("i")
        right = (my + 1) % N_DEV
        left = (my + N_DEV - 1) % N_DEV

        barrier = pltpu.get_barrier_semaphore()
        for nbr in (left, right):
            pl.semaphore_signal(
                barrier, inc=1, device_id=(nbr,),
                device_id_type=pl.DeviceIdType.MESH,
            )

        xcps = {}
        n_xchunk = m_per // X_CHUNK
        for j in range(min(N_XSTAGE, n_xchunk)):
            cp = pltpu.make_async_copy(
                x_hbm.at[pl.ds(j * X_CHUNK, X_CHUNK)], xstage.at[j],
                xsem.at[j],
            )
            cp.start()
            xcps[j] = cp
        for j in range(n_xchunk):
            s = j % N_XSTAGE
            xcps.pop(s).wait()
            x8_ref[pl.ds(j * X_CHUNK, X_CHUNK)] = (
                xstage[s].astype(jnp.float8_e5m2))
            nxt = j + N_XSTAGE
            if nxt < n_xchunk:
                cp = pltpu.make_async_copy(
                    x_hbm.at[pl.ds(nxt * X_CHUNK, X_CHUNK)], xstage.at[s],
                    xsem.at[s],
                )
                cp.start()
                xcps[s] = cp

        pl.semaphore_wait(barrier, 2)

        def send(src, dst, ssem, rsem, dev):
            rdma = pltpu.make_async_remote_copy(
                src_ref=src, dst_ref=dst, send_sem=ssem, recv_sem=rsem,
                device_id=(dev,), device_id_type=pl.DeviceIdType.MESH,
            )
            rdma.start()

        for p in range(N_PIECE):
            send(x8_ref.at[p * q:(p + 1) * q], comm_r.at[0, p * q:(p + 1) * q],
                 ssem_r.at[0, p], rsem_r.at[0, p], right)
        for p in range(N_PIECE):
            send(x8_ref.at[half + p * q:half + (p + 1) * q],
                 comm_l.at[0, p * q:(p + 1) * q],
                 ssem_l.at[0, p], rsem_l.at[0, p], left)

        wcps = {}
        n_wchunk = k // W_CHUNK
        for j in range(min(N_WSTAGE, n_wchunk)):
            cp = pltpu.make_async_copy(
                w_hbm.at[pl.ds(j * W_CHUNK, W_CHUNK)], wstage.at[j % N_WSTAGE],
                wsem.at[j % N_WSTAGE],
            )
            cp.start()
            wcps[j % N_WSTAGE] = cp
        for j in range(n_wchunk):
            s = j % N_WSTAGE
            wcps.pop(s).wait()
            nxt = j + N_WSTAGE
            if nxt < n_wchunk:
                w8_ref[pl.ds(j * W_CHUNK, W_CHUNK)] = (
                    wstage[s].astype(jnp.float8_e5m2))
                cp = pltpu.make_async_copy(
                    w_hbm.at[pl.ds(nxt * W_CHUNK, W_CHUNK)], wstage.at[s],
                    wsem.at[s],
                )
                cp.start()
                wcps[s] = cp
            else:
                w8_ref[pl.ds(j * W_CHUNK, W_CHUNK)] = (
                    wstage[s].astype(jnp.float8_e5m2))

        scale = sx_ref[0] * sw_ref[0]

        pending = {}
        blk = [0]

        def compute(chunk, out_row):
            s = blk[0] % N_STAGE
            blk[0] += 1
            if s in pending:
                pending.pop(s).wait()
            acc = jnp.dot(chunk, w8_ref[...], preferred_element_type=jnp.float32)
            y = acc * scale
            stage[s] = y / (1.0 + jnp.exp(-jnp.clip(y, -60.0, 60.0)))
            cp = pltpu.make_async_copy(
                stage.at[s], out_hbm.at[pl.ds(out_row, q)], osem.at[s]
            )
            cp.start()
            pending[s] = cp

        def compute_own(p):
            compute(x8_ref[p * q:(p + 1) * q], my * m_per + p * q)

        compute_own(0)
        compute_own(1)
        own_next = [2]

        def hop_piece(h, p):
            recv_r = pltpu.make_async_remote_copy(
                src_ref=comm_r.at[h, p * q:(p + 1) * q],
                dst_ref=comm_r.at[h, p * q:(p + 1) * q],
                send_sem=ssem_r.at[h, p], recv_sem=rsem_r.at[h, p],
                device_id=(right,), device_id_type=pl.DeviceIdType.MESH,
            )
            recv_r.wait_recv()
            if h < N_HOP - 1:
                send(comm_r.at[h, p * q:(p + 1) * q],
                     comm_r.at[h + 1, p * q:(p + 1) * q],
                     ssem_r.at[h + 1, p], rsem_r.at[h + 1, p], right)
            recv_l = pltpu.make_async_remote_copy(
                src_ref=comm_l.at[h, p * q:(p + 1) * q],
                dst_ref=comm_l.at[h, p * q:(p + 1) * q],
                send_sem=ssem_l.at[h, p], recv_sem=rsem_l.at[h, p],
                device_id=(left,), device_id_type=pl.DeviceIdType.MESH,
            )
            recv_l.wait_recv()
            if h < N_HOP - 1:
                send(comm_l.at[h, p * q:(p + 1) * q],
                     comm_l.at[h + 1, p * q:(p + 1) * q],
                     ssem_l.at[h + 1, p], rsem_l.at[h + 1, p], left)
            origin_r = (my + N_DEV - 1 - h) % N_DEV
            origin_l = (my + h + 1) % N_DEV
            compute(comm_r[h, p * q:(p + 1) * q], origin_r * m_per + p * q)
            compute(comm_l[h, p * q:(p + 1) * q],
                    origin_l * m_per + half + p * q)

        for h in range(N_HOP):
            for p in range(N_PIECE):
                hop_piece(h, p)
                if own_next[0] < N_PIECE * 2:
                    compute_own(own_next[0])
                    own_next[0] += 1

        for cp in pending.values():
            cp.wait()
        for h in range(N_HOP):
            for p in range(N_PIECE):
                for (sl, ssem, rsem, dev) in (
                    (comm_r, ssem_r, rsem_r, right),
                    (comm_l, ssem_l, rsem_l, left),
                ):
                    done = pltpu.make_async_remote_copy(
                        src_ref=sl.at[h, p * q:(p + 1) * q],
                        dst_ref=sl.at[h, p * q:(p + 1) * q],
                        send_sem=ssem.at[h, p], recv_sem=rsem.at[h, p],
                        device_id=(dev,), device_id_type=pl.DeviceIdType.MESH,
                    )
                    done.wait_send()

    return pl.pallas_call(
        body,
        out_shape=jax.ShapeDtypeStruct((N_DEV * m_per, n_local), jnp.float32),
        in_specs=[
            pl.BlockSpec(memory_space=pl.ANY),
            pl.BlockSpec(memory_space=pl.ANY),
            pl.BlockSpec(memory_space=pltpu.SMEM),
            pl.BlockSpec(memory_space=pltpu.SMEM),
        ],
        out_specs=pl.BlockSpec(memory_space=pl.ANY),
        scratch_shapes=[
            pltpu.VMEM((N_HOP, half, k), jnp.float8_e5m2),
            pltpu.VMEM((N_HOP, half, k), jnp.float8_e5m2),
            pltpu.VMEM((N_STAGE, q, n_local), jnp.float32),
            pltpu.VMEM((k, n_local), jnp.float8_e5m2),
            pltpu.VMEM((2, 512, n_local), jnp.float32),
            pltpu.VMEM((m_per, k), jnp.float8_e5m2),
            pltpu.VMEM((2, 256, k), jnp.float32),
            pltpu.SemaphoreType.DMA((N_HOP, N_PIECE)),
            pltpu.SemaphoreType.DMA((N_HOP, N_PIECE)),
            pltpu.SemaphoreType.DMA((N_HOP, N_PIECE)),
            pltpu.SemaphoreType.DMA((N_HOP, N_PIECE)),
            pltpu.SemaphoreType.DMA((N_STAGE,)),
            pltpu.SemaphoreType.DMA((2,)),
            pltpu.SemaphoreType.DMA((2,)),
        ],
        compiler_params=pltpu.CompilerParams(
            collective_id=0, vmem_limit_bytes=100 * 1024 * 1024
        ),
    )(x, w_mat, scale_x, scale_w)
